# baseline (speedup 1.0000x reference)
import jax
import jax.numpy as jnp
from jax.experimental import pallas as pl
from jax.experimental.pallas import tpu as pltpu

QUEUE_SIZE = 65536
FEATURE_DIM = 128
BATCH = 16384


def _dma_kernel(bf, bl, f, l, of, ol, sem):
    cp0 = pltpu.make_async_copy(bf, of.at[pl.ds(0, BATCH)], sem.at[0])
    cp1 = pltpu.make_async_copy(
        f.at[pl.ds(BATCH, QUEUE_SIZE - BATCH)],
        of.at[pl.ds(BATCH, QUEUE_SIZE - BATCH)], sem.at[1])
    cp2 = pltpu.make_async_copy(bl, ol.at[pl.ds(0, BATCH)], sem.at[2])
    cp3 = pltpu.make_async_copy(
        l.at[pl.ds(BATCH, QUEUE_SIZE - BATCH)],
        ol.at[pl.ds(BATCH, QUEUE_SIZE - BATCH)], sem.at[3])
    cp0.start()
    cp1.start()
    cp2.start()
    cp3.start()
    cp0.wait()
    cp1.wait()
    cp2.wait()
    cp3.wait()


def kernel(batch_features, batch_labels, features, labels):
    out_f, out_l = pl.pallas_call(
        _dma_kernel,
        in_specs=[
            pl.BlockSpec(memory_space=pl.ANY),
            pl.BlockSpec(memory_space=pl.ANY),
            pl.BlockSpec(memory_space=pl.ANY),
            pl.BlockSpec(memory_space=pl.ANY),
        ],
        out_specs=[
            pl.BlockSpec(memory_space=pl.ANY),
            pl.BlockSpec(memory_space=pl.ANY),
        ],
        out_shape=[
            jax.ShapeDtypeStruct((QUEUE_SIZE, FEATURE_DIM), jnp.float32),
            jax.ShapeDtypeStruct((QUEUE_SIZE,), jnp.int32),
        ],
        scratch_shapes=[pltpu.SemaphoreType.DMA((4,))],
    )(batch_features, batch_labels, features, labels)
    return out_f, out_l


# speedup vs baseline: 44.8498x; 44.8498x over previous
import jax
import jax.numpy as jnp
from jax.experimental import pallas as pl
from jax.experimental.pallas import tpu as pltpu

QUEUE_SIZE = 65536
FEATURE_DIM = 128
BATCH = 16384

CHUNK = 2048
NCH = QUEUE_SIZE // CHUNK
NBC = BATCH // CHUNK
NBUF = 8
LOOKAHEAD = 4

LCH = 8192
LNCH = QUEUE_SIZE // LCH
LNBC = BATCH // LCH


def _dma_kernel(bf, bl, f, l, of, ol, buf, lbuf, in_sem, out_sem,
                lin_sem, lout_sem):
    loads = []
    stores = []
    for c in range(NCH):
        src = bf if c < NBC else f
        s = c % NBUF
        loads.append(pltpu.make_async_copy(
            src.at[pl.ds(c * CHUNK, CHUNK)], buf.at[s], in_sem.at[s]))
        stores.append(pltpu.make_async_copy(
            buf.at[s], of.at[pl.ds(c * CHUNK, CHUNK)], out_sem.at[s]))

    lloads = []
    lstores = []
    for c in range(LNCH):
        src = bl if c < LNBC else l
        lloads.append(pltpu.make_async_copy(
            src.at[pl.ds(c * LCH, LCH)], lbuf.at[c], lin_sem.at[c]))
        lstores.append(pltpu.make_async_copy(
            lbuf.at[c], ol.at[pl.ds(c * LCH, LCH)], lout_sem.at[c]))

    for c in range(LNCH):
        lloads[c].start()
    for c in range(LOOKAHEAD):
        loads[c].start()

    for c in range(NCH):
        nxt = c + LOOKAHEAD
        if nxt < NCH:
            if nxt >= NBUF:
                stores[nxt - NBUF].wait()
            loads[nxt].start()
        loads[c].wait()
        stores[c].start()

    for c in range(LNCH):
        lloads[c].wait()
        lstores[c].start()
    for c in range(NCH - NBUF, NCH):
        stores[c].wait()
    for c in range(LNCH):
        lstores[c].wait()


def kernel(batch_features, batch_labels, features, labels):
    out_f, out_l = pl.pallas_call(
        _dma_kernel,
        in_specs=[
            pl.BlockSpec(memory_space=pl.ANY),
            pl.BlockSpec(memory_space=pl.ANY),
            pl.BlockSpec(memory_space=pl.ANY),
            pl.BlockSpec(memory_space=pl.ANY),
        ],
        out_specs=[
            pl.BlockSpec(memory_space=pl.ANY),
            pl.BlockSpec(memory_space=pl.ANY),
        ],
        out_shape=[
            jax.ShapeDtypeStruct((QUEUE_SIZE, FEATURE_DIM), jnp.float32),
            jax.ShapeDtypeStruct((QUEUE_SIZE,), jnp.int32),
        ],
        scratch_shapes=[
            pltpu.VMEM((NBUF, CHUNK, FEATURE_DIM), jnp.float32),
            pltpu.VMEM((LNCH, LCH), jnp.int32),
            pltpu.SemaphoreType.DMA((NBUF,)),
            pltpu.SemaphoreType.DMA((NBUF,)),
            pltpu.SemaphoreType.DMA((LNCH,)),
            pltpu.SemaphoreType.DMA((LNCH,)),
        ],
    )(batch_features, batch_labels, features, labels)
    return out_f, out_l


# speedup vs baseline: 45.5836x; 1.0164x over previous
import jax
import jax.numpy as jnp
from jax.experimental import pallas as pl
from jax.experimental.pallas import tpu as pltpu

QUEUE_SIZE = 65536
FEATURE_DIM = 128
BATCH = 16384

CHUNK = 2048
NCH = QUEUE_SIZE // CHUNK
NBC = BATCH // CHUNK
NBUF = 16
LOOKAHEAD = 8

LCH = 8192
LNCH = QUEUE_SIZE // LCH
LNBC = BATCH // LCH


def _dma_kernel(bf, bl, f, l, of, ol, buf, lbuf, in_sem, out_sem,
                lin_sem, lout_sem):
    loads = []
    stores = []
    for c in range(NCH):
        src = bf if c < NBC else f
        s = c % NBUF
        loads.append(pltpu.make_async_copy(
            src.at[pl.ds(c * CHUNK, CHUNK)], buf.at[s], in_sem.at[s]))
        stores.append(pltpu.make_async_copy(
            buf.at[s], of.at[pl.ds(c * CHUNK, CHUNK)], out_sem.at[s]))

    lloads = []
    lstores = []
    for c in range(LNCH):
        src = bl if c < LNBC else l
        lloads.append(pltpu.make_async_copy(
            src.at[pl.ds(c * LCH, LCH)], lbuf.at[c], lin_sem.at[c]))
        lstores.append(pltpu.make_async_copy(
            lbuf.at[c], ol.at[pl.ds(c * LCH, LCH)], lout_sem.at[c]))

    for c in range(LNCH):
        lloads[c].start()
    for c in range(LOOKAHEAD):
        loads[c].start()

    for c in range(NCH):
        nxt = c + LOOKAHEAD
        if nxt < NCH:
            if nxt >= NBUF:
                stores[nxt - NBUF].wait()
            loads[nxt].start()
        loads[c].wait()
        stores[c].start()

    for c in range(LNCH):
        lloads[c].wait()
        lstores[c].start()
    for c in range(NCH - NBUF, NCH):
        stores[c].wait()
    for c in range(LNCH):
        lstores[c].wait()


def kernel(batch_features, batch_labels, features, labels):
    out_f, out_l = pl.pallas_call(
        _dma_kernel,
        in_specs=[
            pl.BlockSpec(memory_space=pl.ANY),
            pl.BlockSpec(memory_space=pl.ANY),
            pl.BlockSpec(memory_space=pl.ANY),
            pl.BlockSpec(memory_space=pl.ANY),
        ],
        out_specs=[
            pl.BlockSpec(memory_space=pl.ANY),
            pl.BlockSpec(memory_space=pl.ANY),
        ],
        out_shape=[
            jax.ShapeDtypeStruct((QUEUE_SIZE, FEATURE_DIM), jnp.float32),
            jax.ShapeDtypeStruct((QUEUE_SIZE,), jnp.int32),
        ],
        scratch_shapes=[
            pltpu.VMEM((NBUF, CHUNK, FEATURE_DIM), jnp.float32),
            pltpu.VMEM((LNCH, LCH), jnp.int32),
            pltpu.SemaphoreType.DMA((NBUF,)),
            pltpu.SemaphoreType.DMA((NBUF,)),
            pltpu.SemaphoreType.DMA((LNCH,)),
            pltpu.SemaphoreType.DMA((LNCH,)),
        ],
    )(batch_features, batch_labels, features, labels)
    return out_f, out_l
